# baseline (speedup 1.0000x reference)
import functools

import jax
import jax.numpy as jnp
from jax import lax
from jax.experimental import pallas as pl
from jax.experimental.pallas import tpu as pltpu
from jax.experimental.pallas import tpu_sc as plsc

NPG = 1000
K = 500
H = 128
NG = 100
M = NG * K
NSORT = 1024

NC = 2
NS = 16
NW = NC * NS

CHUNK = 128
FULL_CHUNKS = M // CHUNK
REM = M - FULL_CHUNKS * CHUNK
REM_BASE = FULL_CHUNKS * CHUNK
_BASE_PER_W = FULL_CHUNKS // NW
_EXTRA_W = FULL_CHUNKS - _BASE_PER_W * NW


def _lane_sum(a):
    acc = jnp.zeros((a.shape[0], 8), jnp.float32)
    for j in range(16):
        acc = acc + a[:, 8 * j:8 * j + 8]
    q = acc[:, 0:4] + acc[:, 4:8]
    r2 = q[:, 0:2] + q[:, 2:4]
    return r2[:, 0:1] + r2[:, 1:2]


def _score_kernel(x_ref, g_ref, bt_ref, w1_ref, b1_ref, w2_ref, b2_ref,
                  s_ref):
    xb = x_ref[...]
    mu = _lane_sum(xb) * (1.0 / H)
    xc = xb - mu
    var = _lane_sum(xc * xc) * (1.0 / H)
    xn = xc / jnp.sqrt(var + 1e-5) * g_ref[...] + bt_ref[...]
    h = jnp.dot(xn, w1_ref[...], preferred_element_type=jnp.float32)
    h = h + b1_ref[...]
    h = h * (1.0 / (1.0 + jnp.exp(-h)))
    s = jnp.dot(h, w2_ref[...], preferred_element_type=jnp.float32)
    s = s + b2_ref[...]
    s_ref[...] = s.reshape(1, 1, NPG)


def _roll(a, sh):
    return jnp.concatenate([a[:, -sh:], a[:, :-sh]], axis=1)


def _topk_sort_kernel(s_ref, out_ref):
    s = s_ref[...]
    b = lax.bitcast_convert_type(s, jnp.int32)
    key = jnp.where(b < 0, b ^ 0x7FFFFFFF, b)
    pad = jnp.full((NG, NSORT - NPG), jnp.int32(-2147483648))
    key = jnp.concatenate([key, pad], axis=1)
    idx = lax.broadcasted_iota(jnp.int32, (NG, NSORT), 1)
    lane = lax.broadcasted_iota(jnp.int32, (1, NSORT), 1)

    ksize = 2
    while ksize <= NSORT:
        d = ksize // 2
        while d >= 1:
            upper = (lane & d) != 0
            m = ((lane & ksize) == 0) ^ upper
            kp = jnp.where(upper, _roll(key, d), _roll(key, -d))
            ip = jnp.where(upper, _roll(idx, d), _roll(idx, -d))
            win = (key > kp) | ((key == kp) & (idx < ip))
            take_self = win == m
            key = jnp.where(take_self, key, kp)
            idx = jnp.where(take_self, idx, ip)
            d //= 2
        ksize *= 2

    offs = lax.broadcasted_iota(jnp.int32, (NG, K), 0) * NPG
    out_ref[...] = idx[:, :K] + offs


def _topk_indices(x, ln_gamma, ln_beta, W1, b1, W2, b2):
    full = lambda shape: pl.BlockSpec(shape, lambda g: (0,) * len(shape))
    s3 = pl.pallas_call(
        _score_kernel,
        grid=(NG,),
        in_specs=[
            pl.BlockSpec((NPG, H), lambda g: (g, 0)),
            full((1, H)), full((1, H)), full((H, H // 2)),
            full((1, H // 2)), full((H // 2, 1)), full((1, 1)),
        ],
        out_specs=pl.BlockSpec((1, 1, NPG), lambda g: (g, 0, 0)),
        out_shape=jax.ShapeDtypeStruct((NG, 1, NPG), jnp.float32),
    )(x, ln_gamma.reshape(1, H), ln_beta.reshape(1, H), W1,
      b1.reshape(1, H // 2), W2, b2.reshape(1, 1))
    gidx2 = pl.pallas_call(
        _topk_sort_kernel,
        in_specs=[pl.BlockSpec((NG, NPG), lambda: (0, 0))],
        out_specs=pl.BlockSpec((NG, K), lambda: (0, 0)),
        out_shape=jax.ShapeDtypeStruct((NG, K), jnp.int32),
    )(s3.reshape(NG, NPG))
    return gidx2.reshape(M)


@functools.cache
def _make_gather():
    @functools.partial(
        pl.kernel,
        mesh=plsc.VectorSubcoreMesh(core_axis_name="c", subcore_axis_name="s"),
        out_type=jax.ShapeDtypeStruct((M, H), jnp.float32),
        scratch_types=[
            pltpu.VMEM((CHUNK,), jnp.int32),
            pltpu.VMEM((CHUNK, H), jnp.float32),
            pltpu.VMEM((REM,), jnp.int32),
            pltpu.VMEM((REM, H), jnp.float32),
            pltpu.SemaphoreType.DMA,
        ],
    )
    def _gather_rows(x_hbm, idx_hbm, out_hbm, idx_v, rows_v, idx_t, rows_t,
                     sem):
        wid = lax.axis_index("s") * NC + lax.axis_index("c")
        n = jnp.where(wid < _EXTRA_W, _BASE_PER_W + 1, _BASE_PER_W)

        def body(i, carry):
            base = (wid + i * NW) * CHUNK
            pltpu.sync_copy(idx_hbm.at[pl.ds(base, CHUNK)], idx_v)
            pltpu.async_copy(x_hbm.at[idx_v], rows_v, sem).wait()
            pltpu.sync_copy(rows_v, out_hbm.at[pl.ds(base, CHUNK)])
            return carry

        lax.fori_loop(0, n, body, 0)

        @pl.when(wid == NW - 1)
        def _():
            pltpu.sync_copy(idx_hbm.at[pl.ds(REM_BASE, REM)], idx_t)
            pltpu.async_copy(x_hbm.at[idx_t], rows_t, sem).wait()
            pltpu.sync_copy(rows_t, out_hbm.at[pl.ds(REM_BASE, REM)])

    return _gather_rows


def kernel(x, nodes_per_graph, ln_gamma, ln_beta, W1, b1, W2, b2):
    gidx = _topk_indices(x, ln_gamma, ln_beta, W1, b1, W2, b2)
    x_pool = _make_gather()(x, gidx)
    return (x_pool, gidx, K)


# speedup vs baseline: 5.6063x; 5.6063x over previous
import functools

import jax
import jax.numpy as jnp
from jax import lax
from jax.experimental import pallas as pl
from jax.experimental.pallas import tpu as pltpu
from jax.experimental.pallas import tpu_sc as plsc

NPG = 1000
K = 500
H = 128
NG = 100
M = NG * K
NSORT = 1024

RPB = 4000
PADB = 4096
GRID = NG * NPG // RPB

NC = 2
NS = 16
NW = NC * NS

CHUNK = 128
MAIN_PER_W = 12
MAIN_CHUNKS = MAIN_PER_W * NW
EXTRA_CHUNKS = M // CHUNK - MAIN_CHUNKS
REM = M - (M // CHUNK) * CHUNK
REM_BASE = M - REM


def _subl_sum(a):
    acc = jnp.zeros((8, a.shape[1]), jnp.float32)
    for j in range(16):
        acc = acc + a[8 * j:8 * j + 8, :]
    q = acc[0:4, :] + acc[4:8, :]
    r2 = q[0:2, :] + q[2:4, :]
    return r2[0:1, :] + r2[1:2, :]


def _score_kernel(x_ref, g_ref, bt_ref, w1_ref, b1_ref, w2_ref, b2_ref,
                  s_ref):
    xb = x_ref[...]
    xp = jnp.concatenate(
        [xb, jnp.zeros((PADB - RPB, H), jnp.float32)], axis=0)
    xt = xp.T
    mu = _subl_sum(xt) * (1.0 / H)
    xct = xt - mu
    var = _subl_sum(xct * xct) * (1.0 / H)
    xnt = xct / jnp.sqrt(var + 1e-5) * g_ref[...] + bt_ref[...]
    ht = lax.dot_general(w1_ref[...], xnt, (((0,), (0,)), ((), ())),
                         preferred_element_type=jnp.float32)
    ht = ht + b1_ref[...]
    ht = ht * (1.0 / (1.0 + jnp.exp(-ht)))
    st = lax.dot_general(w2_ref[...], ht, (((0,), (0,)), ((), ())),
                         preferred_element_type=jnp.float32)
    st = st + b2_ref[...]
    s_ref[...] = st[:, :RPB].reshape(1, 1, RPB)


def _roll(a, sh):
    return jnp.concatenate([a[:, -sh:], a[:, :-sh]], axis=1)


def _topk_sort_kernel(s_ref, out_ref):
    s = s_ref[...]
    b = lax.bitcast_convert_type(s, jnp.int32)
    key = jnp.where(b < 0, b ^ 0x7FFFFFFF, b)
    pad = jnp.full((NG, NSORT - NPG), jnp.int32(-2147483648))
    key = jnp.concatenate([key, pad], axis=1)
    idx = lax.broadcasted_iota(jnp.int32, (NG, NSORT), 1)
    lane = lax.broadcasted_iota(jnp.int32, (1, NSORT), 1)

    ksize = 2
    while ksize <= NSORT:
        d = ksize // 2
        while d >= 1:
            upper = (lane & d) != 0
            m = ((lane & ksize) == 0) ^ upper
            kp = jnp.where(upper, _roll(key, d), _roll(key, -d))
            ip = jnp.where(upper, _roll(idx, d), _roll(idx, -d))
            win = (key > kp) | ((key == kp) & (idx < ip))
            take_self = win == m
            key = jnp.where(take_self, key, kp)
            idx = jnp.where(take_self, idx, ip)
            d //= 2
        ksize *= 2

    offs = lax.broadcasted_iota(jnp.int32, (NG, K), 0) * NPG
    out_ref[...] = idx[:, :K] + offs


def _topk_indices(x, ln_gamma, ln_beta, W1, b1, W2, b2):
    full = lambda shape: pl.BlockSpec(shape, lambda g: (0,) * len(shape))
    s3 = pl.pallas_call(
        _score_kernel,
        grid=(GRID,),
        in_specs=[
            pl.BlockSpec((RPB, H), lambda g: (g, 0)),
            full((H, 1)), full((H, 1)), full((H, H // 2)),
            full((H // 2, 1)), full((H // 2, 1)), full((1, 1)),
        ],
        out_specs=pl.BlockSpec((1, 1, RPB), lambda g: (g, 0, 0)),
        out_shape=jax.ShapeDtypeStruct((GRID, 1, RPB), jnp.float32),
    )(x, ln_gamma.reshape(H, 1), ln_beta.reshape(H, 1), W1,
      b1.reshape(H // 2, 1), W2, b2.reshape(1, 1))
    gidx2 = pl.pallas_call(
        _topk_sort_kernel,
        in_specs=[pl.BlockSpec((NG, NPG), lambda: (0, 0))],
        out_specs=pl.BlockSpec((NG, K), lambda: (0, 0)),
        out_shape=jax.ShapeDtypeStruct((NG, K), jnp.int32),
    )(s3.reshape(NG, NPG))
    return gidx2.reshape(M)


@functools.cache
def _make_gather():
    @functools.partial(
        pl.kernel,
        mesh=plsc.VectorSubcoreMesh(core_axis_name="c", subcore_axis_name="s"),
        out_type=jax.ShapeDtypeStruct((M, H), jnp.float32),
        scratch_types=[
            pltpu.VMEM((MAIN_PER_W * CHUNK,), jnp.int32),
            pltpu.VMEM((CHUNK, H), jnp.float32),
            pltpu.VMEM((CHUNK, H), jnp.float32),
            pltpu.VMEM((CHUNK,), jnp.int32),
            pltpu.VMEM((CHUNK, H), jnp.float32),
            pltpu.VMEM((REM,), jnp.int32),
            pltpu.VMEM((REM, H), jnp.float32),
            pltpu.SemaphoreType.DMA,
            pltpu.SemaphoreType.DMA,
            pltpu.SemaphoreType.DMA,
            pltpu.SemaphoreType.DMA,
            pltpu.SemaphoreType.DMA,
        ],
    )
    def _gather_rows(x_hbm, idx_hbm, out_hbm, idx_all, r0, r1, idx_e, rows_e,
                     idx_t, rows_t, g0, g1, s0, s1, se):
        wid = lax.axis_index("s") * NC + lax.axis_index("c")
        base_w = wid * (MAIN_PER_W * CHUNK)
        pltpu.sync_copy(idx_hbm.at[pl.ds(base_w, MAIN_PER_W * CHUNK)],
                        idx_all)

        rows = (r0, r1)
        gsem = (g0, g1)
        ssem = (s0, s1)
        hg = [None, None]
        hs = [None, None]
        for i in range(MAIN_PER_W):
            sl = i % 2
            if i >= 2:
                hs[sl].wait()
            hg[sl] = pltpu.async_copy(
                x_hbm.at[idx_all.at[pl.ds(i * CHUNK, CHUNK)]], rows[sl],
                gsem[sl])
            if i >= 1:
                hg[1 - sl].wait()
                hs[1 - sl] = pltpu.async_copy(
                    rows[1 - sl],
                    out_hbm.at[pl.ds(base_w + (i - 1) * CHUNK, CHUNK)],
                    ssem[1 - sl])
        last = (MAIN_PER_W - 1) % 2
        hg[last].wait()
        hs[last] = pltpu.async_copy(
            rows[last],
            out_hbm.at[pl.ds(base_w + (MAIN_PER_W - 1) * CHUNK, CHUNK)],
            ssem[last])
        hs[0].wait()
        hs[1].wait()

        @pl.when(wid < EXTRA_CHUNKS)
        def _():
            base = (MAIN_CHUNKS + wid) * CHUNK
            pltpu.sync_copy(idx_hbm.at[pl.ds(base, CHUNK)], idx_e)
            pltpu.async_copy(x_hbm.at[idx_e], rows_e, se).wait()
            pltpu.sync_copy(rows_e, out_hbm.at[pl.ds(base, CHUNK)])

        @pl.when(wid == EXTRA_CHUNKS)
        def _():
            pltpu.sync_copy(idx_hbm.at[pl.ds(REM_BASE, REM)], idx_t)
            pltpu.async_copy(x_hbm.at[idx_t], rows_t, se).wait()
            pltpu.sync_copy(rows_t, out_hbm.at[pl.ds(REM_BASE, REM)])

    return _gather_rows


def kernel(x, nodes_per_graph, ln_gamma, ln_beta, W1, b1, W2, b2):
    gidx = _topk_indices(x, ln_gamma, ln_beta, W1, b1, W2, b2)
    x_pool = _make_gather()(x, gidx)
    return (x_pool, gidx, K)


# speedup vs baseline: 5.9348x; 1.0586x over previous
import functools

import jax
import jax.numpy as jnp
from jax import lax
from jax.experimental import pallas as pl
from jax.experimental.pallas import tpu as pltpu
from jax.experimental.pallas import tpu_sc as plsc

NPG = 1000
K = 500
H = 128
NG = 100
M = NG * K
NSORT = 1024

RPB = 10000
PADB = 10112
GRID = NG * NPG // RPB

NC = 2
NS = 16
NW = NC * NS

CHUNK = 128
MAIN_PER_W = 12
MAIN_CHUNKS = MAIN_PER_W * NW
EXTRA_CHUNKS = M // CHUNK - MAIN_CHUNKS
REM = M - (M // CHUNK) * CHUNK
REM_BASE = M - REM


def _subl_sum(a):
    acc = jnp.zeros((8, a.shape[1]), jnp.float32)
    for j in range(16):
        acc = acc + a[8 * j:8 * j + 8, :]
    q = acc[0:4, :] + acc[4:8, :]
    r2 = q[0:2, :] + q[2:4, :]
    return r2[0:1, :] + r2[1:2, :]


def _score_kernel(x_ref, g_ref, bt_ref, w1_ref, b1_ref, w2_ref, b2_ref,
                  s_ref):
    xb = x_ref[...]
    xp = jnp.concatenate(
        [xb, jnp.zeros((PADB - RPB, H), jnp.float32)], axis=0)
    xt = xp.T
    mu = _subl_sum(xt) * (1.0 / H)
    xct = xt - mu
    var = _subl_sum(xct * xct) * (1.0 / H)
    xnt = xct / jnp.sqrt(var + 1e-5) * g_ref[...] + bt_ref[...]
    ht = lax.dot_general(w1_ref[...], xnt, (((0,), (0,)), ((), ())),
                         preferred_element_type=jnp.float32)
    ht = ht + b1_ref[...]
    ht = ht * (1.0 / (1.0 + jnp.exp(-ht)))
    st = lax.dot_general(w2_ref[...], ht, (((0,), (0,)), ((), ())),
                         preferred_element_type=jnp.float32)
    st = st + b2_ref[...]
    s_ref[...] = st[:, :RPB].reshape(1, 1, RPB)


def _roll(a, sh):
    return jnp.concatenate([a[:, -sh:], a[:, :-sh]], axis=1)


def _topk_sort_kernel(s_ref, out_ref):
    s = s_ref[...]
    b = lax.bitcast_convert_type(s, jnp.int32)
    key = jnp.where(b < 0, b ^ 0x7FFFFFFF, b)
    pad = jnp.full((NG, NSORT - NPG), jnp.int32(-2147483648))
    key = jnp.concatenate([key, pad], axis=1)
    idx = lax.broadcasted_iota(jnp.int32, (NG, NSORT), 1)
    lane = lax.broadcasted_iota(jnp.int32, (1, NSORT), 1)

    ksize = 2
    while ksize <= NSORT:
        d = ksize // 2
        while d >= 1:
            upper = (lane & d) != 0
            m = ((lane & ksize) == 0) ^ upper
            kp = jnp.where(upper, _roll(key, d), _roll(key, -d))
            ip = jnp.where(upper, _roll(idx, d), _roll(idx, -d))
            win = (key > kp) | ((key == kp) & (idx < ip))
            take_self = win == m
            key = jnp.where(take_self, key, kp)
            idx = jnp.where(take_self, idx, ip)
            d //= 2
        ksize *= 2

    offs = lax.broadcasted_iota(jnp.int32, (NG, K), 0) * NPG
    out_ref[...] = idx[:, :K] + offs


def _topk_indices(x, ln_gamma, ln_beta, W1, b1, W2, b2):
    full = lambda shape: pl.BlockSpec(shape, lambda g: (0,) * len(shape))
    s3 = pl.pallas_call(
        _score_kernel,
        grid=(GRID,),
        in_specs=[
            pl.BlockSpec((RPB, H), lambda g: (g, 0)),
            full((H, 1)), full((H, 1)), full((H, H // 2)),
            full((H // 2, 1)), full((H // 2, 1)), full((1, 1)),
        ],
        out_specs=pl.BlockSpec((1, 1, RPB), lambda g: (g, 0, 0)),
        out_shape=jax.ShapeDtypeStruct((GRID, 1, RPB), jnp.float32),
    )(x, ln_gamma.reshape(H, 1), ln_beta.reshape(H, 1), W1,
      b1.reshape(H // 2, 1), W2, b2.reshape(1, 1))
    gidx2 = pl.pallas_call(
        _topk_sort_kernel,
        in_specs=[pl.BlockSpec((NG, NPG), lambda: (0, 0))],
        out_specs=pl.BlockSpec((NG, K), lambda: (0, 0)),
        out_shape=jax.ShapeDtypeStruct((NG, K), jnp.int32),
    )(s3.reshape(NG, NPG))
    return gidx2.reshape(M)


@functools.cache
def _make_gather():
    @functools.partial(
        pl.kernel,
        mesh=plsc.VectorSubcoreMesh(core_axis_name="c", subcore_axis_name="s"),
        out_type=jax.ShapeDtypeStruct((M, H), jnp.float32),
        scratch_types=[
            pltpu.VMEM((MAIN_PER_W * CHUNK,), jnp.int32),
            pltpu.VMEM((CHUNK, H), jnp.float32),
            pltpu.VMEM((CHUNK, H), jnp.float32),
            pltpu.VMEM((CHUNK,), jnp.int32),
            pltpu.VMEM((CHUNK, H), jnp.float32),
            pltpu.VMEM((REM,), jnp.int32),
            pltpu.VMEM((REM, H), jnp.float32),
            pltpu.SemaphoreType.DMA,
            pltpu.SemaphoreType.DMA,
            pltpu.SemaphoreType.DMA,
            pltpu.SemaphoreType.DMA,
            pltpu.SemaphoreType.DMA,
        ],
    )
    def _gather_rows(x_hbm, idx_hbm, out_hbm, idx_all, r0, r1, idx_e, rows_e,
                     idx_t, rows_t, g0, g1, s0, s1, se):
        wid = lax.axis_index("s") * NC + lax.axis_index("c")
        base_w = wid * (MAIN_PER_W * CHUNK)
        pltpu.sync_copy(idx_hbm.at[pl.ds(base_w, MAIN_PER_W * CHUNK)],
                        idx_all)

        rows = (r0, r1)
        gsem = (g0, g1)
        ssem = (s0, s1)
        hg = [None, None]
        hs = [None, None]
        for i in range(MAIN_PER_W):
            sl = i % 2
            if i >= 2:
                hs[sl].wait()
            hg[sl] = pltpu.async_copy(
                x_hbm.at[idx_all.at[pl.ds(i * CHUNK, CHUNK)]], rows[sl],
                gsem[sl])
            if i >= 1:
                hg[1 - sl].wait()
                hs[1 - sl] = pltpu.async_copy(
                    rows[1 - sl],
                    out_hbm.at[pl.ds(base_w + (i - 1) * CHUNK, CHUNK)],
                    ssem[1 - sl])
        last = (MAIN_PER_W - 1) % 2
        hg[last].wait()
        hs[last] = pltpu.async_copy(
            rows[last],
            out_hbm.at[pl.ds(base_w + (MAIN_PER_W - 1) * CHUNK, CHUNK)],
            ssem[last])
        hs[0].wait()
        hs[1].wait()

        @pl.when(wid < EXTRA_CHUNKS)
        def _():
            base = (MAIN_CHUNKS + wid) * CHUNK
            pltpu.sync_copy(idx_hbm.at[pl.ds(base, CHUNK)], idx_e)
            pltpu.async_copy(x_hbm.at[idx_e], rows_e, se).wait()
            pltpu.sync_copy(rows_e, out_hbm.at[pl.ds(base, CHUNK)])

        @pl.when(wid == EXTRA_CHUNKS)
        def _():
            pltpu.sync_copy(idx_hbm.at[pl.ds(REM_BASE, REM)], idx_t)
            pltpu.async_copy(x_hbm.at[idx_t], rows_t, se).wait()
            pltpu.sync_copy(rows_t, out_hbm.at[pl.ds(REM_BASE, REM)])

    return _gather_rows


def kernel(x, nodes_per_graph, ln_gamma, ln_beta, W1, b1, W2, b2):
    gidx = _topk_indices(x, ln_gamma, ln_beta, W1, b1, W2, b2)
    x_pool = _make_gather()(x, gidx)
    return (x_pool, gidx, K)
